# baseline (device time: 43495 ns/iter reference)
import jax
import jax.numpy as jnp
from jax import lax
from jax.experimental import pallas as pl
from jax.experimental.pallas import tpu as pltpu

CHUNKS = ((0, 128), (128, 384))
NC = len(CHUNKS)


def kernel(x, Win0, Wout0, Win1, Wout1, Win2, Wout2):
    b, d_y = x.shape
    _, h_x = Win0.shape
    bf16 = jnp.bfloat16

    def body(x_ref, win0_ref, wout0_ref, win1_ref, wout1_ref, win2_ref,
             wout2_ref, out_ref,
             sendy_ref, recvy_ref, sendx_ref, recvx_ref,
             ysend_sems, yrecv_sems, xsend_sems, xrecv_sems):
        my_x = lax.axis_index("x")
        my_y = lax.axis_index("y")
        y_peer = (my_x, 1 - my_y)
        x_peer = (1 - my_x, my_y)

        wins = [win0_ref, win1_ref, win2_ref]
        wouts = [wout0_ref, wout1_ref, wout2_ref]
        started = []

        def rdma_y(l, c):
            off, sz = CHUNKS[c]
            return pltpu.make_async_remote_copy(
                src_ref=sendy_ref.at[l, pl.ds(off, sz)],
                dst_ref=recvy_ref.at[l, pl.ds(off, sz)],
                send_sem=ysend_sems.at[l, c],
                recv_sem=yrecv_sems.at[l, c],
                device_id=y_peer,
                device_id_type=pl.DeviceIdType.MESH,
            )

        def rdma_x(l, c):
            off, sz = CHUNKS[c]
            return pltpu.make_async_remote_copy(
                src_ref=sendx_ref.at[l, pl.ds(off, sz)],
                dst_ref=recvx_ref.at[l, pl.ds(off, sz)],
                send_sem=xsend_sems.at[l, c],
                recv_sem=xrecv_sems.at[l, c],
                device_id=x_peer,
                device_id_type=pl.DeviceIdType.MESH,
            )

        barrier_sem = pltpu.get_barrier_semaphore()
        for nbr in [y_peer, x_peer]:
            pl.semaphore_signal(barrier_sem, inc=1, device_id=nbr,
                                device_id_type=pl.DeviceIdType.MESH)

        win_bf = wins[0][...].astype(bf16)
        off0, sz0 = CHUNKS[0]
        x_c0 = x_ref[off0:off0 + sz0, :].astype(bf16)
        p1_c0 = jnp.dot(x_c0, win_bf, preferred_element_type=jnp.float32)
        sendy_ref[0, off0:off0 + sz0, :] = p1_c0.astype(bf16)

        pl.semaphore_wait(barrier_sem, 2)

        r = rdma_y(0, 0)
        r.start()
        started.append(r)
        for c in range(1, NC):
            off, sz = CHUNKS[c]
            x_c = x_ref[off:off + sz, :].astype(bf16)
            p1_c = jnp.dot(x_c, win_bf, preferred_element_type=jnp.float32)
            sendy_ref[0, off:off + sz, :] = p1_c.astype(bf16)
            r = rdma_y(0, c)
            r.start()
            started.append(r)

        for l in range(3):
            wout_bf = wouts[l][...].astype(bf16)
            win_next_bf = wins[l + 1][...].astype(bf16) if l < 2 else None

            for c in range(NC):
                off, sz = CHUNKS[c]
                rdma_y(l, c).wait_recv()
                h_c = jnp.maximum(
                    sendy_ref[l, off:off + sz, :]
                    + recvy_ref[l, off:off + sz, :], 0.0)
                p2_c = jnp.dot(h_c, wout_bf, preferred_element_type=jnp.float32)
                sendx_ref[l, off:off + sz, :] = p2_c.astype(bf16)
                r = rdma_x(l, c)
                r.start()
                started.append(r)

            for c in range(NC):
                off, sz = CHUNKS[c]
                rdma_x(l, c).wait_recv()
                if l < 2:
                    x_c = (sendx_ref[l, off:off + sz, :]
                           + recvx_ref[l, off:off + sz, :])
                    p1_c = jnp.dot(x_c, win_next_bf,
                                   preferred_element_type=jnp.float32)
                    sendy_ref[l + 1, off:off + sz, :] = p1_c.astype(bf16)
                    r = rdma_y(l + 1, c)
                    r.start()
                    started.append(r)
                else:
                    out_ref[off:off + sz, :] = (
                        sendx_ref[l, off:off + sz, :].astype(jnp.float32)
                        + recvx_ref[l, off:off + sz, :].astype(jnp.float32)
                    )

        for r in started:
            r.wait_send()

    return pl.pallas_call(
        body,
        out_shape=jax.ShapeDtypeStruct((b, d_y), jnp.float32),
        in_specs=[pl.BlockSpec(memory_space=pltpu.VMEM)] * 7,
        out_specs=pl.BlockSpec(memory_space=pltpu.VMEM),
        scratch_shapes=[
            pltpu.VMEM((3, b, h_x), bf16),
            pltpu.VMEM((3, b, h_x), bf16),
            pltpu.VMEM((3, b, d_y), bf16),
            pltpu.VMEM((3, b, d_y), bf16),
            pltpu.SemaphoreType.DMA((3, NC)),
            pltpu.SemaphoreType.DMA((3, NC)),
            pltpu.SemaphoreType.DMA((3, NC)),
            pltpu.SemaphoreType.DMA((3, NC)),
        ],
        compiler_params=pltpu.CompilerParams(collective_id=0),
    )(x, Win0, Wout0, Win1, Wout1, Win2, Wout2)


# device time: 38326 ns/iter; 1.1349x vs baseline; 1.1349x over previous
import jax
import jax.numpy as jnp
from jax import lax
from jax.experimental import pallas as pl
from jax.experimental.pallas import tpu as pltpu

NC = 2


def kernel(x, Win0, Wout0, Win1, Wout1, Win2, Wout2):
    b, d_y = x.shape
    _, h_x = Win0.shape
    ch = b // NC
    bf16 = jnp.bfloat16

    def body(x_ref, win0_ref, wout0_ref, win1_ref, wout1_ref, win2_ref,
             wout2_ref, out_ref,
             sendy_ref, recvy_ref, sendx_ref, recvx_ref,
             ysend_sems, yrecv_sems, xsend_sems, xrecv_sems):
        my_x = lax.axis_index("x")
        my_y = lax.axis_index("y")
        y_peer = (my_x, 1 - my_y)
        x_peer = (1 - my_x, my_y)

        wins = [win0_ref, win1_ref, win2_ref]
        wouts = [wout0_ref, wout1_ref, wout2_ref]
        started = []

        def rdma_y(l, c):
            return pltpu.make_async_remote_copy(
                src_ref=sendy_ref.at[l, c],
                dst_ref=recvy_ref.at[l, c],
                send_sem=ysend_sems.at[l, c],
                recv_sem=yrecv_sems.at[l, c],
                device_id=y_peer,
                device_id_type=pl.DeviceIdType.MESH,
            )

        def rdma_x(l, c):
            return pltpu.make_async_remote_copy(
                src_ref=sendx_ref.at[l, c],
                dst_ref=recvx_ref.at[l, c],
                send_sem=xsend_sems.at[l, c],
                recv_sem=xrecv_sems.at[l, c],
                device_id=x_peer,
                device_id_type=pl.DeviceIdType.MESH,
            )

        barrier_sem = pltpu.get_barrier_semaphore()
        for nbr in [y_peer, x_peer]:
            pl.semaphore_signal(barrier_sem, inc=1, device_id=nbr,
                                device_id_type=pl.DeviceIdType.MESH)

        win_bf = wins[0][...].astype(bf16)
        x_c = x_ref[0:ch, :].astype(bf16)
        p1_c = jnp.dot(x_c, win_bf, preferred_element_type=jnp.float32)
        sendy_ref[0, 0] = p1_c.astype(bf16)

        pl.semaphore_wait(barrier_sem, 2)

        r = rdma_y(0, 0)
        r.start()
        started.append(r)
        for c in range(1, NC):
            x_c = x_ref[c * ch:(c + 1) * ch, :].astype(bf16)
            p1_c = jnp.dot(x_c, win_bf, preferred_element_type=jnp.float32)
            sendy_ref[0, c] = p1_c.astype(bf16)
            r = rdma_y(0, c)
            r.start()
            started.append(r)

        for l in range(3):
            wout_bf = wouts[l][...].astype(bf16)
            win_next_bf = wins[l + 1][...].astype(bf16) if l < 2 else None

            for c in range(NC):
                rdma_y(l, c).wait_recv()
                h_c = jnp.maximum(sendy_ref[l, c] + recvy_ref[l, c], 0.0)
                p2_c = jnp.dot(h_c, wout_bf, preferred_element_type=jnp.float32)
                sendx_ref[l, c] = p2_c.astype(bf16)
                r = rdma_x(l, c)
                r.start()
                started.append(r)

            for c in range(NC):
                rdma_x(l, c).wait_recv()
                if l < 2:
                    x_c = sendx_ref[l, c] + recvx_ref[l, c]
                    p1_c = jnp.dot(x_c, win_next_bf,
                                   preferred_element_type=jnp.float32)
                    sendy_ref[l + 1, c] = p1_c.astype(bf16)
                    r = rdma_y(l + 1, c)
                    r.start()
                    started.append(r)
                else:
                    out_ref[c * ch:(c + 1) * ch, :] = (
                        sendx_ref[l, c].astype(jnp.float32)
                        + recvx_ref[l, c].astype(jnp.float32)
                    )

        for r in started:
            r.wait_send()

    return pl.pallas_call(
        body,
        out_shape=jax.ShapeDtypeStruct((b, d_y), jnp.float32),
        in_specs=[pl.BlockSpec(memory_space=pltpu.VMEM)] * 7,
        out_specs=pl.BlockSpec(memory_space=pltpu.VMEM),
        scratch_shapes=[
            pltpu.VMEM((3, NC, ch, h_x), bf16),
            pltpu.VMEM((3, NC, ch, h_x), bf16),
            pltpu.VMEM((3, NC, ch, d_y), bf16),
            pltpu.VMEM((3, NC, ch, d_y), bf16),
            pltpu.SemaphoreType.DMA((3, NC)),
            pltpu.SemaphoreType.DMA((3, NC)),
            pltpu.SemaphoreType.DMA((3, NC)),
            pltpu.SemaphoreType.DMA((3, NC)),
        ],
        compiler_params=pltpu.CompilerParams(collective_id=0),
    )(x, Win0, Wout0, Win1, Wout1, Win2, Wout2)


# device time: 35272 ns/iter; 1.2331x vs baseline; 1.0866x over previous
import jax
import jax.numpy as jnp
from jax import lax
from jax.experimental import pallas as pl
from jax.experimental.pallas import tpu as pltpu

NC = 2


def kernel(x, Win0, Wout0, Win1, Wout1, Win2, Wout2):
    b, d_y = x.shape
    _, h_x = Win0.shape
    ch = b // NC
    bf16 = jnp.bfloat16

    def body(x_ref, win0_ref, wout0_ref, win1_ref, wout1_ref, win2_ref,
             wout2_ref, out_ref,
             sendy0_ref, recvy0_ref, sendx_ref, recvx_ref,
             agsend_ref, agrecv_ref, winsend_ref, winrecv_ref, p1own_ref,
             y0send_sems, y0recv_sems, xsend_sems, xrecv_sems,
             agsend_sems, agrecv_sems, wsend_sems, wrecv_sems):
        my_x = lax.axis_index("x")
        my_y = lax.axis_index("y")
        y_peer = (my_x, 1 - my_y)
        x_peer = (1 - my_x, my_y)

        started = []

        def rdma_y0(c):
            return pltpu.make_async_remote_copy(
                src_ref=sendy0_ref.at[c], dst_ref=recvy0_ref.at[c],
                send_sem=y0send_sems.at[c], recv_sem=y0recv_sems.at[c],
                device_id=y_peer, device_id_type=pl.DeviceIdType.MESH,
            )

        def rdma_x(l, c):
            return pltpu.make_async_remote_copy(
                src_ref=sendx_ref.at[l, c], dst_ref=recvx_ref.at[l, c],
                send_sem=xsend_sems.at[l, c], recv_sem=xrecv_sems.at[l, c],
                device_id=x_peer, device_id_type=pl.DeviceIdType.MESH,
            )

        def rdma_ag(l, c):
            return pltpu.make_async_remote_copy(
                src_ref=agsend_ref.at[l, c], dst_ref=agrecv_ref.at[l, c],
                send_sem=agsend_sems.at[l, c], recv_sem=agrecv_sems.at[l, c],
                device_id=y_peer, device_id_type=pl.DeviceIdType.MESH,
            )

        def rdma_win(i):
            return pltpu.make_async_remote_copy(
                src_ref=winsend_ref.at[i], dst_ref=winrecv_ref.at[i],
                send_sem=wsend_sems.at[i], recv_sem=wrecv_sems.at[i],
                device_id=y_peer, device_id_type=pl.DeviceIdType.MESH,
            )

        barrier_sem = pltpu.get_barrier_semaphore()
        for nbr in [y_peer, x_peer]:
            pl.semaphore_signal(barrier_sem, inc=1, device_id=nbr,
                                device_id_type=pl.DeviceIdType.MESH)

        win0_bf = win0_ref[...].astype(bf16)
        x_c = x_ref[0:ch, :].astype(bf16)
        p1_c = jnp.dot(x_c, win0_bf, preferred_element_type=jnp.float32)
        sendy0_ref[0] = p1_c.astype(bf16)

        pl.semaphore_wait(barrier_sem, 2)

        r = rdma_y0(0)
        r.start()
        started.append(r)
        for c in range(1, NC):
            x_c = x_ref[c * ch:(c + 1) * ch, :].astype(bf16)
            p1_c = jnp.dot(x_c, win0_bf, preferred_element_type=jnp.float32)
            sendy0_ref[c] = p1_c.astype(bf16)
            r = rdma_y0(c)
            r.start()
            started.append(r)

        winsend_ref[0] = win1_ref[...].astype(bf16)
        r = rdma_win(0)
        r.start()
        started.append(r)

        wout_bf = wout0_ref[...].astype(bf16)

        for c in range(NC):
            rdma_y0(c).wait_recv()
            h_c = jnp.maximum(sendy0_ref[c] + recvy0_ref[c], 0.0)
            p2_c = jnp.dot(h_c, wout_bf, preferred_element_type=jnp.float32)
            sendx_ref[0, c] = p2_c.astype(bf16)
            r = rdma_x(0, c)
            r.start()
            started.append(r)

        winsend_ref[1] = win2_ref[...].astype(bf16)

        wins_own = [None, winsend_ref.at[0], winsend_ref.at[1]]
        wouts = [None, wout1_ref, wout2_ref]

        for l in range(2):
            for c in range(NC):
                rdma_x(l, c).wait_recv()
                x_c = sendx_ref[l, c] + recvx_ref[l, c]
                agsend_ref[l, c] = x_c
                r = rdma_ag(l, c)
                r.start()
                started.append(r)
                p1own_ref[c] = jnp.dot(x_c, wins_own[l + 1][...],
                                       preferred_element_type=jnp.float32)

            if l == 0:
                r = rdma_win(1)
                r.start()
                started.append(r)
            wout_bf = wouts[l + 1][...].astype(bf16)
            rdma_win(l).wait_recv()

            for c in range(NC):
                rdma_ag(l, c).wait_recv()
                p1_full = p1own_ref[c] + jnp.dot(
                    agrecv_ref[l, c], winrecv_ref[l],
                    preferred_element_type=jnp.float32)
                h_c = jnp.maximum(p1_full, 0.0).astype(bf16)
                p2_c = jnp.dot(h_c, wout_bf, preferred_element_type=jnp.float32)
                sendx_ref[l + 1, c] = p2_c.astype(bf16)
                r = rdma_x(l + 1, c)
                r.start()
                started.append(r)

        for c in range(NC):
            rdma_x(2, c).wait_recv()
            out_ref[c * ch:(c + 1) * ch, :] = (
                sendx_ref[2, c].astype(jnp.float32)
                + recvx_ref[2, c].astype(jnp.float32)
            )

        for r in started:
            r.wait_send()

    return pl.pallas_call(
        body,
        out_shape=jax.ShapeDtypeStruct((b, d_y), jnp.float32),
        in_specs=[pl.BlockSpec(memory_space=pltpu.VMEM)] * 7,
        out_specs=pl.BlockSpec(memory_space=pltpu.VMEM),
        scratch_shapes=[
            pltpu.VMEM((NC, ch, h_x), bf16),
            pltpu.VMEM((NC, ch, h_x), bf16),
            pltpu.VMEM((3, NC, ch, d_y), bf16),
            pltpu.VMEM((3, NC, ch, d_y), bf16),
            pltpu.VMEM((2, NC, ch, d_y), bf16),
            pltpu.VMEM((2, NC, ch, d_y), bf16),
            pltpu.VMEM((2, d_y, h_x), bf16),
            pltpu.VMEM((2, d_y, h_x), bf16),
            pltpu.VMEM((NC, ch, h_x), jnp.float32),
            pltpu.SemaphoreType.DMA((NC,)),
            pltpu.SemaphoreType.DMA((NC,)),
            pltpu.SemaphoreType.DMA((3, NC)),
            pltpu.SemaphoreType.DMA((3, NC)),
            pltpu.SemaphoreType.DMA((2, NC)),
            pltpu.SemaphoreType.DMA((2, NC)),
            pltpu.SemaphoreType.DMA((2,)),
            pltpu.SemaphoreType.DMA((2,)),
        ],
        compiler_params=pltpu.CompilerParams(collective_id=0),
    )(x, Win0, Wout0, Win1, Wout1, Win2, Wout2)
